# Initial kernel scaffold; baseline (speedup 1.0000x reference)
import jax
import jax.numpy as jnp
from jax.experimental import pallas as pl


def kernel(x, params, edge_index, gate):
    raise NotImplementedError("write your pallas kernel here")


# speedup vs baseline: 4.3046x; 4.3046x over previous
import functools

import jax
import jax.numpy as jnp
from jax import lax
from jax.experimental import pallas as pl
from jax.experimental.pallas import tpu as pltpu
from jax.experimental.pallas import tpu_sc as plsc

_NS = 16
_CHUNK = 80
_RB = 80
_BLK = 2000


def _tc_premlp(x, w2, b2):
    N, D = x.shape

    def body(x_ref, w_ref, b_ref, o_ref):
        acc = lax.dot_general(x_ref[...], w_ref[0],
                              (((1,), (0,)), ((), ())),
                              preferred_element_type=jnp.float32)
        o_ref[0] = jnp.maximum(acc + b_ref[0], 0.0)

    return pl.pallas_call(
        body,
        grid=(2, N // _BLK),
        in_specs=[
            pl.BlockSpec((_BLK, D), lambda k, i: (i, 0)),
            pl.BlockSpec((1, D, D), lambda k, i: (k, 0, 0)),
            pl.BlockSpec((1, 1, D), lambda k, i: (k, 0, 0)),
        ],
        out_specs=pl.BlockSpec((1, _BLK, D), lambda k, i: (k, i, 0)),
        out_shape=jax.ShapeDtypeStruct((2, N, D), jnp.float32),
    )(x, w2, b2)


def _sc_segment_sum(y2n, src, dst, zeros_nd):
    twoN, D = y2n.shape
    N = twoN // 2
    E = src.shape[0]
    ept = E // _NS
    nchunk = ept // _CHUNK
    nrb = N // _RB
    rit = (nrb + _NS - 1) // _NS
    mesh = plsc.VectorSubcoreMesh(core_axis_name="c", subcore_axis_name="s")

    @functools.partial(
        pl.kernel,
        out_type=jax.ShapeDtypeStruct((twoN, D), jnp.float32),
        mesh=mesh,
        scratch_types=[
            pltpu.VMEM((_CHUNK,), jnp.int32),
            pltpu.VMEM((_CHUNK,), jnp.int32),
            pltpu.VMEM((_CHUNK, D), jnp.float32),
            pltpu.VMEM_SHARED((N, D), jnp.float32),
            pltpu.SemaphoreType.DMA,
        ],
    )
    def k(y_hbm, src_hbm, dst_hbm, z_hbm, out_hbm, src_v, dst_v, rows_v,
          acc, sem):
        c = lax.axis_index("c")
        s = lax.axis_index("s")
        shift = c * N

        def zstep(i, carry):
            blk = s + i * _NS

            @pl.when(blk < nrb)
            def _():
                off = pl.multiple_of(blk * _RB, 8)
                pltpu.sync_copy(z_hbm.at[pl.ds(off, _RB)],
                                acc.at[pl.ds(off, _RB)])
            return carry

        lax.fori_loop(0, rit, zstep, 0)
        plsc.subcore_barrier()
        base = s * ept

        def step(i, carry):
            off = pl.multiple_of(base + i * _CHUNK, 8)
            pltpu.sync_copy(src_hbm.at[pl.ds(off, _CHUNK)], src_v)
            pltpu.sync_copy(dst_hbm.at[pl.ds(off, _CHUNK)], dst_v)
            for j in range(_CHUNK // 16):
                sl = pl.ds(j * 16, 16)
                src_v[sl] = src_v[sl] + shift
            pltpu.async_copy(y_hbm.at[src_v], rows_v, sem).wait()
            pltpu.sync_copy(rows_v, acc.at[dst_v], add=True)
            return carry

        lax.fori_loop(0, nchunk, step, 0)
        plsc.subcore_barrier()

        def wstep(i, carry):
            blk = s + i * _NS

            @pl.when(blk < nrb)
            def _():
                off = pl.multiple_of(blk * _RB, 8)
                pltpu.sync_copy(acc.at[pl.ds(off, _RB)],
                                out_hbm.at[pl.ds(pl.multiple_of(
                                    shift + blk * _RB, 8), _RB)])
            return carry

        lax.fori_loop(0, rit, wstep, 0)

    return k(y2n, src, dst, zeros_nd)


def _tc_post(agg, gate2d, wih2, consts, wr1, wr2, wr3):
    _, N, D = agg.shape

    def body(a_ref, g_ref, wih_ref, c_ref, w1_ref, w2_ref, w3_ref, o_ref):
        cst = c_ref[...]

        def gru(a, widx, b):
            gi = lax.dot_general(a, wih_ref[widx], (((1,), (0,)), ((), ())),
                                 preferred_element_type=jnp.float32)
            r = jax.nn.sigmoid(gi[:, :D] + cst[b])
            z = jax.nn.sigmoid(gi[:, D:2 * D] + cst[b + 1])
            n = jnp.tanh(gi[:, 2 * D:] + cst[b + 2] + r * cst[b + 3])
            return (1.0 - z) * n + z * cst[8]

        hf_af = gru(a_ref[0], 0, 0)
        hf_nf = gru(a_ref[1], 1, 4)
        g = g_ref[...]
        hf = jnp.where(g == 1, hf_af,
                       jnp.where(g == 2, hf_nf, cst[8][None]))
        mm = lambda u, w: lax.dot_general(u, w, (((1,), (0,)), ((), ())),
                                          preferred_element_type=jnp.float32)
        h1 = jnp.maximum(mm(hf, w1_ref[...]) + cst[9], 0.0)
        h2 = jnp.maximum(mm(h1, w2_ref[...]) + cst[10], 0.0)
        pv = mm(h2, w3_ref[...]) + cst[11]
        o_ref[...] = pv[:, :1]

    return pl.pallas_call(
        body,
        grid=(N // _BLK,),
        in_specs=[
            pl.BlockSpec((2, _BLK, D), lambda i: (0, i, 0)),
            pl.BlockSpec((_BLK, 1), lambda i: (i, 0)),
            pl.BlockSpec((2, D, 3 * D), lambda i: (0, 0, 0)),
            pl.BlockSpec((16, D), lambda i: (0, 0)),
            pl.BlockSpec((D, D), lambda i: (0, 0)),
            pl.BlockSpec((D, D), lambda i: (0, 0)),
            pl.BlockSpec((D, D), lambda i: (0, 0)),
        ],
        out_specs=pl.BlockSpec((_BLK, 1), lambda i: (i, 0)),
        out_shape=jax.ShapeDtypeStruct((N, 1), jnp.float32),
    )(agg, gate2d, wih2, consts, wr1, wr2, wr3)


def kernel(x, params, edge_index, gate):
    p = params
    N, D = x.shape
    DM = p['W_r1'].shape[1]
    f32 = jnp.float32

    hf0 = p['W_hf'][0] + p['b_hf']
    c_af = hf0 @ p['W_af'][D:] + p['b_af']
    w2 = jnp.stack([p['W_af'][:D], p['W_nf']])
    b2 = jnp.stack([c_af, p['b_nf']])[:, None, :]

    def gru_consts(name):
        gh = hf0 @ p['Whh_' + name] + p['bhh_' + name]
        bih = p['bih_' + name]
        return [bih[:D] + gh[:D], bih[D:2 * D] + gh[D:2 * D],
                bih[2 * D:], gh[2 * D:]]

    zrow = jnp.zeros((D,), f32)
    br1 = zrow.at[:DM].set(p['b_r1'])
    br2 = zrow.at[:DM].set(p['b_r2'])
    br3 = zrow.at[0].set(p['b_r3'][0])
    consts = jnp.stack(gru_consts('af') + gru_consts('nf')
                       + [hf0, br1, br2, br3] + [zrow] * 4)
    wih2 = jnp.stack([p['Wih_af'], p['Wih_nf']])
    wr1 = jnp.zeros((D, D), f32).at[:, :DM].set(p['W_r1'])
    wr2 = jnp.zeros((D, D), f32).at[:DM, :DM].set(p['W_r2'])
    wr3 = jnp.zeros((D, D), f32).at[:DM, :1].set(p['W_r3'])

    y = _tc_premlp(x, w2, b2).reshape(2 * N, D)
    agg = _sc_segment_sum(y, edge_index[0], edge_index[1],
                          jnp.zeros((N, D), f32)).reshape(2, N, D)
    return _tc_post(agg, gate.reshape(N, 1), wih2, consts, wr1, wr2, wr3)


# speedup vs baseline: 9.7842x; 2.2729x over previous
import functools

import jax
import jax.numpy as jnp
from jax import lax
from jax.experimental import pallas as pl
from jax.experimental.pallas import tpu as pltpu
from jax.experimental.pallas import tpu_sc as plsc

_NS = 16
_CHUNK = 80
_EBLK = 4000
_RB = 80
_BLK = 2000


def _tc_premlp(x, w2, b2):
    N, D = x.shape

    def body(x_ref, w_ref, b_ref, o_ref):
        acc = lax.dot_general(x_ref[...], w_ref[0],
                              (((1,), (0,)), ((), ())),
                              preferred_element_type=jnp.float32)
        o_ref[0] = jnp.maximum(acc + b_ref[0], 0.0)

    return pl.pallas_call(
        body,
        grid=(2, N // _BLK),
        in_specs=[
            pl.BlockSpec((_BLK, D), lambda k, i: (i, 0)),
            pl.BlockSpec((1, D, D), lambda k, i: (k, 0, 0)),
            pl.BlockSpec((1, 1, D), lambda k, i: (k, 0, 0)),
        ],
        out_specs=pl.BlockSpec((1, _BLK, D), lambda k, i: (k, i, 0)),
        out_shape=jax.ShapeDtypeStruct((2, N, D), jnp.float32),
    )(x, w2, b2)


def _sc_segment_sum(y2, src, dst, zeros_nd):
    _, N, D = y2.shape
    E = src.shape[0]
    ept = E // _NS
    nblk = ept // _EBLK
    bchunk = _EBLK // _CHUNK
    nrb = N // _RB
    rit = (nrb + _NS - 1) // _NS
    mesh = plsc.VectorSubcoreMesh(core_axis_name="c", subcore_axis_name="s")

    @functools.partial(
        pl.kernel,
        out_type=jax.ShapeDtypeStruct((2, N, D), jnp.float32),
        mesh=mesh,
        scratch_types=[
            pltpu.VMEM((_EBLK,), jnp.int32),
            pltpu.VMEM((_EBLK,), jnp.int32),
            pltpu.VMEM((_CHUNK, D), jnp.float32),
            pltpu.VMEM((_CHUNK, D), jnp.float32),
            pltpu.VMEM_SHARED((N, D), jnp.float32),
            pltpu.SemaphoreType.DMA,
            pltpu.SemaphoreType.DMA,
        ],
    )
    def k(y_hbm, src_hbm, dst_hbm, z_hbm, out_hbm, src_v, dst_v,
          rows0, rows1, acc, sem0, sem1):
        c = lax.axis_index("c")
        s = lax.axis_index("s")
        y_h = y_hbm.at[c]
        rows = (rows0, rows1)
        sems = (sem0, sem1)

        def zstep(i, carry):
            blk = s + i * _NS

            @pl.when(blk < nrb)
            def _():
                off = pl.multiple_of(blk * _RB, 8)
                pltpu.sync_copy(z_hbm.at[pl.ds(off, _RB)],
                                acc.at[pl.ds(off, _RB)])
            return carry

        lax.fori_loop(0, rit, zstep, 0)
        plsc.subcore_barrier()

        def blk_body(j, carry):
            base = pl.multiple_of((s * nblk + j) * _EBLK, 8)
            pltpu.sync_copy(src_hbm.at[pl.ds(base, _EBLK)], src_v)
            pltpu.sync_copy(dst_hbm.at[pl.ds(base, _EBLK)], dst_v)

            for b in range(2):
                pltpu.async_copy(
                    y_h.at[src_v.at[pl.ds(b * _CHUNK, _CHUNK)]], rows[b],
                    sems[b])

            def step(g, carry2):
                for b in range(2):
                    i = g * 2 + b
                    off = pl.multiple_of(i * _CHUNK, 8)
                    pltpu.make_async_copy(y_h.at[pl.ds(0, _CHUNK)],
                                          rows[b], sems[b]).wait()
                    pltpu.sync_copy(rows[b],
                                    acc.at[dst_v.at[pl.ds(off, _CHUNK)]],
                                    add=True)

                    @pl.when(i + 2 < bchunk)
                    def _():
                        off2 = pl.multiple_of((i + 2) * _CHUNK, 8)
                        pltpu.async_copy(
                            y_h.at[src_v.at[pl.ds(off2, _CHUNK)]],
                            rows[b], sems[b])
                return carry2

            lax.fori_loop(0, bchunk // 2, step, 0)
            return carry

        lax.fori_loop(0, nblk, blk_body, 0)
        plsc.subcore_barrier()
        o_h = out_hbm.at[c]

        def wstep(i, carry):
            blk = s + i * _NS

            @pl.when(blk < nrb)
            def _():
                off = pl.multiple_of(blk * _RB, 8)
                pltpu.sync_copy(acc.at[pl.ds(off, _RB)],
                                o_h.at[pl.ds(off, _RB)])
            return carry

        lax.fori_loop(0, rit, wstep, 0)

    return k(y2, src, dst, zeros_nd)


def _tc_post(agg, gate2d, wih2, consts, wr1, wr2, wr3):
    _, N, D = agg.shape

    def body(a_ref, g_ref, wih_ref, c_ref, w1_ref, w2_ref, w3_ref, o_ref):
        cst = c_ref[...]

        def gru(a, widx, b):
            gi = lax.dot_general(a, wih_ref[widx], (((1,), (0,)), ((), ())),
                                 preferred_element_type=jnp.float32)
            r = jax.nn.sigmoid(gi[:, :D] + cst[b])
            z = jax.nn.sigmoid(gi[:, D:2 * D] + cst[b + 1])
            n = jnp.tanh(gi[:, 2 * D:] + cst[b + 2] + r * cst[b + 3])
            return (1.0 - z) * n + z * cst[8]

        hf_af = gru(a_ref[0], 0, 0)
        hf_nf = gru(a_ref[1], 1, 4)
        g = g_ref[...]
        hf = jnp.where(g == 1, hf_af,
                       jnp.where(g == 2, hf_nf, cst[8][None]))
        mm = lambda u, w: lax.dot_general(u, w, (((1,), (0,)), ((), ())),
                                          preferred_element_type=jnp.float32)
        h1 = jnp.maximum(mm(hf, w1_ref[...]) + cst[9], 0.0)
        h2 = jnp.maximum(mm(h1, w2_ref[...]) + cst[10], 0.0)
        pv = mm(h2, w3_ref[...]) + cst[11]
        o_ref[...] = pv[:, :1]

    return pl.pallas_call(
        body,
        grid=(N // _BLK,),
        in_specs=[
            pl.BlockSpec((2, _BLK, D), lambda i: (0, i, 0)),
            pl.BlockSpec((_BLK, 1), lambda i: (i, 0)),
            pl.BlockSpec((2, D, 3 * D), lambda i: (0, 0, 0)),
            pl.BlockSpec((16, D), lambda i: (0, 0)),
            pl.BlockSpec((D, D), lambda i: (0, 0)),
            pl.BlockSpec((D, D), lambda i: (0, 0)),
            pl.BlockSpec((D, D), lambda i: (0, 0)),
        ],
        out_specs=pl.BlockSpec((_BLK, 1), lambda i: (i, 0)),
        out_shape=jax.ShapeDtypeStruct((N, 1), jnp.float32),
    )(agg, gate2d, wih2, consts, wr1, wr2, wr3)


def kernel(x, params, edge_index, gate):
    p = params
    N, D = x.shape
    DM = p['W_r1'].shape[1]
    f32 = jnp.float32

    hf0 = p['W_hf'][0] + p['b_hf']
    c_af = hf0 @ p['W_af'][D:] + p['b_af']
    w2 = jnp.stack([p['W_af'][:D], p['W_nf']])
    b2 = jnp.stack([c_af, p['b_nf']])[:, None, :]

    def gru_consts(name):
        gh = hf0 @ p['Whh_' + name] + p['bhh_' + name]
        bih = p['bih_' + name]
        return [bih[:D] + gh[:D], bih[D:2 * D] + gh[D:2 * D],
                bih[2 * D:], gh[2 * D:]]

    zrow = jnp.zeros((D,), f32)
    br1 = zrow.at[:DM].set(p['b_r1'])
    br2 = zrow.at[:DM].set(p['b_r2'])
    br3 = zrow.at[0].set(p['b_r3'][0])
    consts = jnp.stack(gru_consts('af') + gru_consts('nf')
                       + [hf0, br1, br2, br3] + [zrow] * 4)
    wih2 = jnp.stack([p['Wih_af'], p['Wih_nf']])
    wr1 = jnp.zeros((D, D), f32).at[:, :DM].set(p['W_r1'])
    wr2 = jnp.zeros((D, D), f32).at[:DM, :DM].set(p['W_r2'])
    wr3 = jnp.zeros((D, D), f32).at[:DM, :1].set(p['W_r3'])

    y = _tc_premlp(x, w2, b2)
    agg = _sc_segment_sum(y, edge_index[0], edge_index[1],
                          jnp.zeros((N, D), f32))
    return _tc_post(agg, gate.reshape(N, 1), wih2, consts, wr1, wr2, wr3)


# speedup vs baseline: 10.6158x; 1.0850x over previous
import functools

import jax
import jax.numpy as jnp
from jax import lax
from jax.experimental import pallas as pl
from jax.experimental.pallas import tpu as pltpu
from jax.experimental.pallas import tpu_sc as plsc

_NS = 16
_CHUNK = 160
_EBLK = 4000
_RB = 80
_BLK = 2000


def _tc_premlp(x, w2, b2):
    N, D = x.shape

    def body(x_ref, w_ref, b_ref, o_ref):
        acc = lax.dot_general(x_ref[...], w_ref[0],
                              (((1,), (0,)), ((), ())),
                              preferred_element_type=jnp.float32)
        o_ref[0] = jnp.maximum(acc + b_ref[0], 0.0)

    return pl.pallas_call(
        body,
        grid=(2, N // _BLK),
        in_specs=[
            pl.BlockSpec((_BLK, D), lambda k, i: (i, 0)),
            pl.BlockSpec((1, D, D), lambda k, i: (k, 0, 0)),
            pl.BlockSpec((1, 1, D), lambda k, i: (k, 0, 0)),
        ],
        out_specs=pl.BlockSpec((1, _BLK, D), lambda k, i: (k, i, 0)),
        out_shape=jax.ShapeDtypeStruct((2, N, D), jnp.float32),
    )(x, w2, b2)


def _sc_segment_sum(y2, src, dst, zeros_nd):
    _, N, D = y2.shape
    E = src.shape[0]
    ept = E // _NS
    nblk = ept // _EBLK
    bchunk = _EBLK // _CHUNK
    nrb = N // _RB
    rit = (nrb + _NS - 1) // _NS
    mesh = plsc.VectorSubcoreMesh(core_axis_name="c", subcore_axis_name="s")

    @functools.partial(
        pl.kernel,
        out_type=jax.ShapeDtypeStruct((2, N, D), jnp.float32),
        mesh=mesh,
        scratch_types=[
            pltpu.VMEM((_EBLK,), jnp.int32),
            pltpu.VMEM((_EBLK,), jnp.int32),
            pltpu.VMEM((_CHUNK, D), jnp.float32),
            pltpu.VMEM((_CHUNK, D), jnp.float32),
            pltpu.VMEM_SHARED((N, D), jnp.float32),
            pltpu.SemaphoreType.DMA,
            pltpu.SemaphoreType.DMA,
        ],
    )
    def k(y_hbm, src_hbm, dst_hbm, z_hbm, out_hbm, src_v, dst_v,
          rows0, rows1, acc, sem0, sem1):
        c = lax.axis_index("c")
        s = lax.axis_index("s")
        y_h = y_hbm.at[c]
        rows = (rows0, rows1)
        sems = (sem0, sem1)

        def zstep(i, carry):
            blk = s + i * _NS

            @pl.when(blk < nrb)
            def _():
                off = pl.multiple_of(blk * _RB, 8)
                pltpu.sync_copy(z_hbm.at[pl.ds(off, _RB)],
                                acc.at[pl.ds(off, _RB)])
            return carry

        lax.fori_loop(0, rit, zstep, 0)
        plsc.subcore_barrier()

        def blk_body(j, carry):
            base = pl.multiple_of((s * nblk + j) * _EBLK, 8)
            pltpu.sync_copy(src_hbm.at[pl.ds(base, _EBLK)], src_v)
            pltpu.sync_copy(dst_hbm.at[pl.ds(base, _EBLK)], dst_v)

            for b in range(2):
                pltpu.async_copy(
                    y_h.at[src_v.at[pl.ds(b * _CHUNK, _CHUNK)]], rows[b],
                    sems[b])

            def step(g, carry2):
                for b in range(2):
                    i = g * 2 + b
                    off = pl.multiple_of(i * _CHUNK, 8)
                    pltpu.make_async_copy(y_h.at[pl.ds(0, _CHUNK)],
                                          rows[b], sems[b]).wait()
                    pltpu.sync_copy(rows[b],
                                    acc.at[dst_v.at[pl.ds(off, _CHUNK)]],
                                    add=True)

                    @pl.when(i + 2 < bchunk)
                    def _():
                        off2 = pl.multiple_of((i + 2) * _CHUNK, 8)
                        pltpu.async_copy(
                            y_h.at[src_v.at[pl.ds(off2, _CHUNK)]],
                            rows[b], sems[b])
                return carry2

            lax.fori_loop(0, bchunk // 2, step, 0)
            if bchunk % 2 == 1:
                i = bchunk - 1
                off = pl.multiple_of(i * _CHUNK, 8)
                pltpu.make_async_copy(y_h.at[pl.ds(0, _CHUNK)],
                                      rows[0], sems[0]).wait()
                pltpu.sync_copy(rows[0],
                                acc.at[dst_v.at[pl.ds(off, _CHUNK)]],
                                add=True)
            return carry

        lax.fori_loop(0, nblk, blk_body, 0)
        plsc.subcore_barrier()
        o_h = out_hbm.at[c]

        def wstep(i, carry):
            blk = s + i * _NS

            @pl.when(blk < nrb)
            def _():
                off = pl.multiple_of(blk * _RB, 8)
                pltpu.sync_copy(acc.at[pl.ds(off, _RB)],
                                o_h.at[pl.ds(off, _RB)])
            return carry

        lax.fori_loop(0, rit, wstep, 0)

    return k(y2, src, dst, zeros_nd)


def _tc_post(agg, gate2d, wih2, consts, wr1, wr2, wr3):
    _, N, D = agg.shape

    def body(a_ref, g_ref, wih_ref, c_ref, w1_ref, w2_ref, w3_ref, o_ref):
        cst = c_ref[...]

        def gru(a, widx, b):
            gi = lax.dot_general(a, wih_ref[widx], (((1,), (0,)), ((), ())),
                                 preferred_element_type=jnp.float32)
            r = jax.nn.sigmoid(gi[:, :D] + cst[b])
            z = jax.nn.sigmoid(gi[:, D:2 * D] + cst[b + 1])
            n = jnp.tanh(gi[:, 2 * D:] + cst[b + 2] + r * cst[b + 3])
            return (1.0 - z) * n + z * cst[8]

        hf_af = gru(a_ref[0], 0, 0)
        hf_nf = gru(a_ref[1], 1, 4)
        g = g_ref[...]
        hf = jnp.where(g == 1, hf_af,
                       jnp.where(g == 2, hf_nf, cst[8][None]))
        mm = lambda u, w: lax.dot_general(u, w, (((1,), (0,)), ((), ())),
                                          preferred_element_type=jnp.float32)
        h1 = jnp.maximum(mm(hf, w1_ref[...]) + cst[9], 0.0)
        h2 = jnp.maximum(mm(h1, w2_ref[...]) + cst[10], 0.0)
        pv = mm(h2, w3_ref[...]) + cst[11]
        o_ref[...] = pv[:, :1]

    return pl.pallas_call(
        body,
        grid=(N // _BLK,),
        in_specs=[
            pl.BlockSpec((2, _BLK, D), lambda i: (0, i, 0)),
            pl.BlockSpec((_BLK, 1), lambda i: (i, 0)),
            pl.BlockSpec((2, D, 3 * D), lambda i: (0, 0, 0)),
            pl.BlockSpec((16, D), lambda i: (0, 0)),
            pl.BlockSpec((D, D), lambda i: (0, 0)),
            pl.BlockSpec((D, D), lambda i: (0, 0)),
            pl.BlockSpec((D, D), lambda i: (0, 0)),
        ],
        out_specs=pl.BlockSpec((_BLK, 1), lambda i: (i, 0)),
        out_shape=jax.ShapeDtypeStruct((N, 1), jnp.float32),
    )(agg, gate2d, wih2, consts, wr1, wr2, wr3)


def kernel(x, params, edge_index, gate):
    p = params
    N, D = x.shape
    DM = p['W_r1'].shape[1]
    f32 = jnp.float32

    hf0 = p['W_hf'][0] + p['b_hf']
    c_af = hf0 @ p['W_af'][D:] + p['b_af']
    w2 = jnp.stack([p['W_af'][:D], p['W_nf']])
    b2 = jnp.stack([c_af, p['b_nf']])[:, None, :]

    def gru_consts(name):
        gh = hf0 @ p['Whh_' + name] + p['bhh_' + name]
        bih = p['bih_' + name]
        return [bih[:D] + gh[:D], bih[D:2 * D] + gh[D:2 * D],
                bih[2 * D:], gh[2 * D:]]

    zrow = jnp.zeros((D,), f32)
    br1 = zrow.at[:DM].set(p['b_r1'])
    br2 = zrow.at[:DM].set(p['b_r2'])
    br3 = zrow.at[0].set(p['b_r3'][0])
    consts = jnp.stack(gru_consts('af') + gru_consts('nf')
                       + [hf0, br1, br2, br3] + [zrow] * 4)
    wih2 = jnp.stack([p['Wih_af'], p['Wih_nf']])
    wr1 = jnp.zeros((D, D), f32).at[:, :DM].set(p['W_r1'])
    wr2 = jnp.zeros((D, D), f32).at[:DM, :DM].set(p['W_r2'])
    wr3 = jnp.zeros((D, D), f32).at[:DM, :1].set(p['W_r3'])

    y = _tc_premlp(x, w2, b2)
    agg = _sc_segment_sum(y, edge_index[0], edge_index[1],
                          jnp.zeros((N, D), f32))
    return _tc_post(agg, gate.reshape(N, 1), wih2, consts, wr1, wr2, wr3)


# speedup vs baseline: 11.5441x; 1.0875x over previous
import functools

import jax
import jax.numpy as jnp
from jax import lax
from jax.experimental import pallas as pl
from jax.experimental.pallas import tpu as pltpu
from jax.experimental.pallas import tpu_sc as plsc

_NS = 16
_CHUNK = 80
_NBUF = 4
_EBLK = 4000
_RB = 80
_BLK = 2000


def _tc_premlp(x, w2, b2):
    N, D = x.shape

    def body(x_ref, w_ref, b_ref, o_ref):
        acc = lax.dot_general(x_ref[...], w_ref[0],
                              (((1,), (0,)), ((), ())),
                              preferred_element_type=jnp.float32)
        o_ref[0] = jnp.maximum(acc + b_ref[0], 0.0)

    return pl.pallas_call(
        body,
        grid=(2, N // _BLK),
        in_specs=[
            pl.BlockSpec((_BLK, D), lambda k, i: (i, 0)),
            pl.BlockSpec((1, D, D), lambda k, i: (k, 0, 0)),
            pl.BlockSpec((1, 1, D), lambda k, i: (k, 0, 0)),
        ],
        out_specs=pl.BlockSpec((1, _BLK, D), lambda k, i: (k, i, 0)),
        out_shape=jax.ShapeDtypeStruct((2, N, D), jnp.float32),
    )(x, w2, b2)


def _sc_segment_sum(y2, src, dst, zeros_nd):
    _, N, D = y2.shape
    E = src.shape[0]
    ept = E // _NS
    nblk = ept // _EBLK
    bchunk = _EBLK // _CHUNK
    nrb = N // _RB
    rit = (nrb + _NS - 1) // _NS
    mesh = plsc.VectorSubcoreMesh(core_axis_name="c", subcore_axis_name="s")

    @functools.partial(
        pl.kernel,
        out_type=jax.ShapeDtypeStruct((2, N, D), jnp.float32),
        mesh=mesh,
        scratch_types=[
            pltpu.VMEM((_EBLK,), jnp.int32),
            pltpu.VMEM((_EBLK,), jnp.int32),
            pltpu.VMEM((_CHUNK, D), jnp.float32),
            pltpu.VMEM((_CHUNK, D), jnp.float32),
            pltpu.VMEM((_CHUNK, D), jnp.float32),
            pltpu.VMEM((_CHUNK, D), jnp.float32),
            pltpu.VMEM_SHARED((N, D), jnp.float32),
            pltpu.SemaphoreType.DMA,
            pltpu.SemaphoreType.DMA,
            pltpu.SemaphoreType.DMA,
            pltpu.SemaphoreType.DMA,
        ],
    )
    def k(y_hbm, src_hbm, dst_hbm, z_hbm, out_hbm, src_v, dst_v,
          rows0, rows1, rows2, rows3, acc, sem0, sem1, sem2, sem3):
        c = lax.axis_index("c")
        s = lax.axis_index("s")
        y_h = y_hbm.at[c]
        rows = (rows0, rows1, rows2, rows3)
        sems = (sem0, sem1, sem2, sem3)

        def zstep(i, carry):
            blk = s + i * _NS

            @pl.when(blk < nrb)
            def _():
                off = pl.multiple_of(blk * _RB, 8)
                pltpu.sync_copy(z_hbm.at[pl.ds(off, _RB)],
                                acc.at[pl.ds(off, _RB)])
            return carry

        lax.fori_loop(0, rit, zstep, 0)
        plsc.subcore_barrier()

        def blk_body(j, carry):
            base = pl.multiple_of((s * nblk + j) * _EBLK, 8)
            pltpu.sync_copy(src_hbm.at[pl.ds(base, _EBLK)], src_v)
            pltpu.sync_copy(dst_hbm.at[pl.ds(base, _EBLK)], dst_v)

            for b in range(_NBUF):
                pltpu.async_copy(
                    y_h.at[src_v.at[pl.ds(b * _CHUNK, _CHUNK)]], rows[b],
                    sems[b])

            def step(g, carry2):
                for b in range(_NBUF):
                    i = g * _NBUF + b
                    off = pl.multiple_of(i * _CHUNK, 8)
                    pltpu.make_async_copy(y_h.at[pl.ds(0, _CHUNK)],
                                          rows[b], sems[b]).wait()
                    pltpu.sync_copy(rows[b],
                                    acc.at[dst_v.at[pl.ds(off, _CHUNK)]],
                                    add=True)

                    @pl.when(i + _NBUF < bchunk)
                    def _():
                        off2 = pl.multiple_of((i + _NBUF) * _CHUNK, 8)
                        pltpu.async_copy(
                            y_h.at[src_v.at[pl.ds(off2, _CHUNK)]],
                            rows[b], sems[b])
                return carry2

            nfull = bchunk // _NBUF
            lax.fori_loop(0, nfull, step, 0)
            for b in range(bchunk - nfull * _NBUF):
                i = nfull * _NBUF + b
                off = pl.multiple_of(i * _CHUNK, 8)
                pltpu.make_async_copy(y_h.at[pl.ds(0, _CHUNK)],
                                      rows[b], sems[b]).wait()
                pltpu.sync_copy(rows[b],
                                acc.at[dst_v.at[pl.ds(off, _CHUNK)]],
                                add=True)
            return carry

        lax.fori_loop(0, nblk, blk_body, 0)
        plsc.subcore_barrier()
        o_h = out_hbm.at[c]

        def wstep(i, carry):
            blk = s + i * _NS

            @pl.when(blk < nrb)
            def _():
                off = pl.multiple_of(blk * _RB, 8)
                pltpu.sync_copy(acc.at[pl.ds(off, _RB)],
                                o_h.at[pl.ds(off, _RB)])
            return carry

        lax.fori_loop(0, rit, wstep, 0)

    return k(y2, src, dst, zeros_nd)


def _tc_post(agg, gate2d, wih2, consts, wr1, wr2, wr3):
    _, N, D = agg.shape

    def body(a_ref, g_ref, wih_ref, c_ref, w1_ref, w2_ref, w3_ref, o_ref):
        cst = c_ref[...]

        def gru(a, widx, b):
            gi = lax.dot_general(a, wih_ref[widx], (((1,), (0,)), ((), ())),
                                 preferred_element_type=jnp.float32)
            r = jax.nn.sigmoid(gi[:, :D] + cst[b])
            z = jax.nn.sigmoid(gi[:, D:2 * D] + cst[b + 1])
            n = jnp.tanh(gi[:, 2 * D:] + cst[b + 2] + r * cst[b + 3])
            return (1.0 - z) * n + z * cst[8]

        hf_af = gru(a_ref[0], 0, 0)
        hf_nf = gru(a_ref[1], 1, 4)
        g = g_ref[...]
        hf = jnp.where(g == 1, hf_af,
                       jnp.where(g == 2, hf_nf, cst[8][None]))
        mm = lambda u, w: lax.dot_general(u, w, (((1,), (0,)), ((), ())),
                                          preferred_element_type=jnp.float32)
        h1 = jnp.maximum(mm(hf, w1_ref[...]) + cst[9], 0.0)
        h2 = jnp.maximum(mm(h1, w2_ref[...]) + cst[10], 0.0)
        pv = mm(h2, w3_ref[...]) + cst[11]
        o_ref[...] = pv[:, :1]

    return pl.pallas_call(
        body,
        grid=(N // _BLK,),
        in_specs=[
            pl.BlockSpec((2, _BLK, D), lambda i: (0, i, 0)),
            pl.BlockSpec((_BLK, 1), lambda i: (i, 0)),
            pl.BlockSpec((2, D, 3 * D), lambda i: (0, 0, 0)),
            pl.BlockSpec((16, D), lambda i: (0, 0)),
            pl.BlockSpec((D, D), lambda i: (0, 0)),
            pl.BlockSpec((D, D), lambda i: (0, 0)),
            pl.BlockSpec((D, D), lambda i: (0, 0)),
        ],
        out_specs=pl.BlockSpec((_BLK, 1), lambda i: (i, 0)),
        out_shape=jax.ShapeDtypeStruct((N, 1), jnp.float32),
    )(agg, gate2d, wih2, consts, wr1, wr2, wr3)


def kernel(x, params, edge_index, gate):
    p = params
    N, D = x.shape
    DM = p['W_r1'].shape[1]
    f32 = jnp.float32

    hf0 = p['W_hf'][0] + p['b_hf']
    c_af = hf0 @ p['W_af'][D:] + p['b_af']
    w2 = jnp.stack([p['W_af'][:D], p['W_nf']])
    b2 = jnp.stack([c_af, p['b_nf']])[:, None, :]

    def gru_consts(name):
        gh = hf0 @ p['Whh_' + name] + p['bhh_' + name]
        bih = p['bih_' + name]
        return [bih[:D] + gh[:D], bih[D:2 * D] + gh[D:2 * D],
                bih[2 * D:], gh[2 * D:]]

    zrow = jnp.zeros((D,), f32)
    br1 = zrow.at[:DM].set(p['b_r1'])
    br2 = zrow.at[:DM].set(p['b_r2'])
    br3 = zrow.at[0].set(p['b_r3'][0])
    consts = jnp.stack(gru_consts('af') + gru_consts('nf')
                       + [hf0, br1, br2, br3] + [zrow] * 4)
    wih2 = jnp.stack([p['Wih_af'], p['Wih_nf']])
    wr1 = jnp.zeros((D, D), f32).at[:, :DM].set(p['W_r1'])
    wr2 = jnp.zeros((D, D), f32).at[:DM, :DM].set(p['W_r2'])
    wr3 = jnp.zeros((D, D), f32).at[:DM, :1].set(p['W_r3'])

    y = _tc_premlp(x, w2, b2)
    agg = _sc_segment_sum(y, edge_index[0], edge_index[1],
                          jnp.zeros((N, D), f32))
    return _tc_post(agg, gate.reshape(N, 1), wih2, consts, wr1, wr2, wr3)
